# baseline (speedup 1.0000x reference)
import functools

import jax
import jax.numpy as jnp
from jax import lax
from jax.experimental import pallas as pl
from jax.experimental.pallas import tpu as pltpu
from jax.experimental.pallas import tpu_sc as plsc

N_PTS = 50000
CIN = 128
COUT = 128
K_OFF = 9

NC = 2
NS = 16
ROWS_PER_SC = 25000
ACC_ROWS = 25088
STRIPE = ACC_ROWS // NS
DUMMY = ROWS_PER_SC
CHUNK = 128
SUP = 8
SUP_PAIRS = SUP * CHUNK
PAIRS_PAD = 458752
SUPS_PER_TILE = PAIRS_PAD // (NS * SUP_PAIRS)

BN_MM = 2000
BN_EW = 2000


def _mm_body(x_ref, w_ref, y_ref):
    x = x_ref[...].astype(jnp.bfloat16)
    w = w_ref[0].astype(jnp.bfloat16)
    y = jnp.dot(x, w, preferred_element_type=jnp.float32)
    y_ref[0] = y.astype(jnp.bfloat16)


def _dense_matmuls(feats, weight):
    nb = N_PTS // BN_MM
    return pl.pallas_call(
        _mm_body,
        grid=(nb, K_OFF),
        in_specs=[
            pl.BlockSpec((BN_MM, CIN), lambda n, k: (n, 0)),
            pl.BlockSpec((1, CIN, COUT), lambda n, k: (k, 0, 0)),
        ],
        out_specs=pl.BlockSpec((1, BN_MM, COUT), lambda n, k: (k, n, 0)),
        out_shape=jax.ShapeDtypeStruct((K_OFF, N_PTS, COUT), jnp.bfloat16),
    )(feats, weight)


def _sc_scatter(y, in_flat, out_flat, zrs):
    mesh = plsc.VectorSubcoreMesh(core_axis_name="c", subcore_axis_name="s")

    @functools.partial(
        pl.kernel,
        mesh=mesh,
        compiler_params=pltpu.CompilerParams(use_tc_tiling_on_sc=False),
        out_type=jax.ShapeDtypeStruct((N_PTS, COUT), jnp.bfloat16),
        scratch_types=[
            pltpu.VMEM((SUP_PAIRS,), jnp.int32),
            pltpu.VMEM((SUP_PAIRS,), jnp.int32),
            pltpu.VMEM((SUP, CHUNK), jnp.int32),
            pltpu.VMEM((2 * CHUNK, COUT), jnp.bfloat16),
            pltpu.VMEM_SHARED((ACC_ROWS, COUT), jnp.bfloat16),
            pltpu.SemaphoreType.DMA,
            pltpu.SemaphoreType.DMA,
        ],
    )
    def scatter_kernel(y_h, inf_h, outf_h, zrs_h, o_h,
                       idx_v, oraw_v, loc_v, rows_v, acc_s, sem0, sem1):
        cid = lax.axis_index("c")
        sid = lax.axis_index("s")
        lo = cid * ROWS_PER_SC

        pltpu.sync_copy(zrs_h, acc_s.at[pl.ds(sid * STRIPE, STRIPE)])
        plsc.subcore_barrier()

        def sup_body(ci, _):
            base = (sid * SUPS_PER_TILE + ci) * SUP_PAIRS
            pltpu.sync_copy(inf_h.at[pl.ds(base, SUP_PAIRS)], idx_v)
            pltpu.sync_copy(outf_h.at[pl.ds(base, SUP_PAIRS)], oraw_v)
            for jr in range(SUP):
                for jc in range(CHUNK // 16):
                    o = oraw_v[pl.ds(jr * CHUNK + jc * 16, 16)]
                    keep = (o >= lo) & (o < lo + ROWS_PER_SC)
                    loc_v[jr, pl.ds(jc * 16, 16)] = (
                        jnp.where(keep, o - lo, DUMMY))

            def gather(j):
                slot = (j % 2) * CHUNK
                return pltpu.async_copy(
                    y_h.at[idx_v.at[pl.ds(j * CHUNK, CHUNK)]],
                    rows_v.at[pl.ds(slot, CHUNK)],
                    sem0 if j % 2 == 0 else sem1)

            cp = gather(0)
            for j in range(SUP):
                nxt = gather(j + 1) if j + 1 < SUP else None
                cp.wait()
                pltpu.sync_copy(
                    rows_v.at[pl.ds((j % 2) * CHUNK, CHUNK)],
                    acc_s.at[loc_v.at[j]], add=True)
                cp = nxt
            return 0

        lax.fori_loop(0, SUPS_PER_TILE, sup_body, 0)
        plsc.subcore_barrier()

        @pl.when(sid < NS - 1)
        def _():
            pltpu.sync_copy(
                acc_s.at[pl.ds(sid * STRIPE, STRIPE)],
                o_h.at[pl.ds(lo + sid * STRIPE, STRIPE)])

        @pl.when(sid == NS - 1)
        def _():
            tail = ROWS_PER_SC - (NS - 1) * STRIPE
            pltpu.sync_copy(
                acc_s.at[pl.ds((NS - 1) * STRIPE, tail)],
                o_h.at[pl.ds(lo + (NS - 1) * STRIPE, tail)])

    return scatter_kernel(y, in_flat, out_flat, zrs)


def _stats_body(x_ref, s_ref, q_ref):
    x = x_ref[...].astype(jnp.float32)
    s = jnp.sum(x, axis=0, keepdims=True)
    q = jnp.sum(x * x, axis=0, keepdims=True)

    @pl.when(pl.program_id(0) == 0)
    def _():
        s_ref[...] = s
        q_ref[...] = q

    @pl.when(pl.program_id(0) != 0)
    def _():
        s_ref[...] += s
        q_ref[...] += q


def _channel_stats(acc):
    nb = N_PTS // BN_EW
    one = pl.BlockSpec((1, COUT), lambda n: (0, 0))
    return pl.pallas_call(
        _stats_body,
        grid=(nb,),
        in_specs=[pl.BlockSpec((BN_EW, COUT), lambda n: (n, 0))],
        out_specs=[one, one],
        out_shape=[jax.ShapeDtypeStruct((1, COUT), jnp.float32)] * 2,
    )(acc)


def _norm_body(x_ref, a_ref, b_ref, o_ref):
    x = x_ref[...].astype(jnp.float32)
    o_ref[...] = jnp.maximum(x * a_ref[...] + b_ref[...], 0.0)


def _normalize(acc, a, b):
    nb = N_PTS // BN_EW
    return pl.pallas_call(
        _norm_body,
        grid=(nb,),
        in_specs=[
            pl.BlockSpec((BN_EW, COUT), lambda n: (n, 0)),
            pl.BlockSpec((1, COUT), lambda n: (0, 0)),
            pl.BlockSpec((1, COUT), lambda n: (0, 0)),
        ],
        out_specs=pl.BlockSpec((BN_EW, COUT), lambda n: (n, 0)),
        out_shape=jax.ShapeDtypeStruct((N_PTS, COUT), jnp.float32),
    )(acc, a, b)


def kernel(coords, feats, rules, weight, bias, gamma, beta):
    y = _dense_matmuls(feats, weight)
    y = y.reshape(K_OFF * N_PTS, COUT)

    offs = (jnp.arange(K_OFF, dtype=jnp.int32) * N_PTS)[:, None]
    in_flat = (rules[:, 0, :] + offs).reshape(-1)
    out_flat = rules[:, 1, :].reshape(-1)
    pad = PAIRS_PAD - in_flat.shape[0]
    in_flat = jnp.pad(in_flat, (0, pad))
    out_flat = jnp.pad(out_flat, (0, pad), constant_values=N_PTS)
    zrs = jnp.zeros((STRIPE, COUT), jnp.bfloat16)

    acc = _sc_scatter(y, in_flat, out_flat, zrs)

    s, q = _channel_stats(acc)
    mean = s / N_PTS
    var = q / N_PTS - mean * mean
    a = (gamma / jnp.sqrt(var[0] + 1e-5))[None]
    b = (beta + (bias - mean[0]) * a[0])[None]
    out = _normalize(acc, a, b)
    return (coords, out)


# speedup vs baseline: 1.2993x; 1.2993x over previous
import functools

import jax
import jax.numpy as jnp
from jax import lax
from jax.experimental import pallas as pl
from jax.experimental.pallas import tpu as pltpu
from jax.experimental.pallas import tpu_sc as plsc

N_PTS = 50000
CIN = 128
COUT = 128
K_OFF = 9

NC = 2
NS = 16
ROWS_PER_SC = 25000
ACC_ROWS = 25088
STRIPE = ACC_ROWS // NS
DUMMY = ROWS_PER_SC
CHUNK = 128
SUP = 8
SUP_PAIRS = SUP * CHUNK
PAIRS_REAL = K_OFF * N_PTS
PAIRS_SPAN = 458752
SUPS_PER_TILE = PAIRS_SPAN // (NS * SUP_PAIRS)

BN_MM = 2000
BN_EW = 2000


def _mm_body(x_ref, w_ref, y_ref):
    y = jnp.dot(x_ref[...], w_ref[0], preferred_element_type=jnp.float32)
    y_ref[0] = y.astype(jnp.bfloat16)


def _dense_matmuls(feats, weight):
    nb = N_PTS // BN_MM
    return pl.pallas_call(
        _mm_body,
        grid=(nb, K_OFF),
        in_specs=[
            pl.BlockSpec((BN_MM, CIN), lambda n, k: (n, 0)),
            pl.BlockSpec((1, CIN, COUT), lambda n, k: (k, 0, 0)),
        ],
        out_specs=pl.BlockSpec((1, BN_MM, COUT), lambda n, k: (k, n, 0)),
        out_shape=jax.ShapeDtypeStruct((K_OFF, N_PTS, COUT), jnp.bfloat16),
    )(feats, weight)


def _sc_scatter(y, in_flat, out_flat, zrs):
    mesh = plsc.VectorSubcoreMesh(core_axis_name="c", subcore_axis_name="s")

    @functools.partial(
        pl.kernel,
        mesh=mesh,
        compiler_params=pltpu.CompilerParams(use_tc_tiling_on_sc=False),
        out_type=jax.ShapeDtypeStruct((N_PTS, COUT), jnp.bfloat16),
        scratch_types=[
            pltpu.VMEM((SUP_PAIRS,), jnp.int32),
            pltpu.VMEM((SUP_PAIRS,), jnp.int32),
            pltpu.VMEM((SUP, CHUNK), jnp.int32),
            pltpu.VMEM((2 * CHUNK, COUT), jnp.bfloat16),
            pltpu.VMEM_SHARED((ACC_ROWS, COUT), jnp.bfloat16),
            pltpu.SemaphoreType.DMA,
            pltpu.SemaphoreType.DMA,
        ],
    )
    def scatter_kernel(y_h, inf_h, outf_h, zrs_h, o_h,
                       idx_v, oraw_v, loc_v, rows_v, acc_s, sem0, sem1):
        cid = lax.axis_index("c")
        sid = lax.axis_index("s")
        lo = cid * ROWS_PER_SC

        pltpu.sync_copy(zrs_h, acc_s.at[pl.ds(sid * STRIPE, STRIPE)])
        plsc.subcore_barrier()

        lane = lax.iota(jnp.int32, 16)

        def sup_body(ci, _):
            b_n = (sid * SUPS_PER_TILE + ci) * SUP_PAIRS
            b_r = pl.multiple_of(
                jnp.minimum(b_n, PAIRS_REAL - SUP_PAIRS), 8)
            diff = b_n - b_r
            pltpu.sync_copy(inf_h.at[pl.ds(b_r, SUP_PAIRS)], idx_v)
            pltpu.sync_copy(outf_h.at[pl.ds(b_r, SUP_PAIRS)], oraw_v)
            for jr in range(SUP):
                for jc in range(CHUNK // 16):
                    off = jr * CHUNK + jc * 16
                    o = oraw_v[pl.ds(off, 16)]
                    keep = ((o >= lo) & (o < lo + ROWS_PER_SC)
                            & (lane + off >= diff))
                    loc_v[jr, pl.ds(jc * 16, 16)] = (
                        jnp.where(keep, o - lo, DUMMY))

            def gather(j):
                slot = (j % 2) * CHUNK
                return pltpu.async_copy(
                    y_h.at[idx_v.at[pl.ds(j * CHUNK, CHUNK)]],
                    rows_v.at[pl.ds(slot, CHUNK)],
                    sem0 if j % 2 == 0 else sem1)

            cp = gather(0)
            for j in range(SUP):
                nxt = gather(j + 1) if j + 1 < SUP else None
                cp.wait()
                pltpu.sync_copy(
                    rows_v.at[pl.ds((j % 2) * CHUNK, CHUNK)],
                    acc_s.at[loc_v.at[j]], add=True)
                cp = nxt
            return 0

        lax.fori_loop(0, SUPS_PER_TILE, sup_body, 0)
        plsc.subcore_barrier()

        @pl.when(sid < NS - 1)
        def _():
            pltpu.sync_copy(
                acc_s.at[pl.ds(sid * STRIPE, STRIPE)],
                o_h.at[pl.ds(lo + sid * STRIPE, STRIPE)])

        @pl.when(sid == NS - 1)
        def _():
            tail = ROWS_PER_SC - (NS - 1) * STRIPE
            pltpu.sync_copy(
                acc_s.at[pl.ds((NS - 1) * STRIPE, tail)],
                o_h.at[pl.ds(lo + (NS - 1) * STRIPE, tail)])

    return scatter_kernel(y, in_flat, out_flat, zrs)


def _stats_body(x_ref, s_ref, q_ref):
    x = x_ref[...].astype(jnp.float32)
    s = jnp.sum(x, axis=0, keepdims=True)
    q = jnp.sum(x * x, axis=0, keepdims=True)

    @pl.when(pl.program_id(0) == 0)
    def _():
        s_ref[...] = s
        q_ref[...] = q

    @pl.when(pl.program_id(0) != 0)
    def _():
        s_ref[...] += s
        q_ref[...] += q


def _channel_stats(acc):
    nb = N_PTS // BN_EW
    one = pl.BlockSpec((1, COUT), lambda n: (0, 0))
    return pl.pallas_call(
        _stats_body,
        grid=(nb,),
        in_specs=[pl.BlockSpec((BN_EW, COUT), lambda n: (n, 0))],
        out_specs=[one, one],
        out_shape=[jax.ShapeDtypeStruct((1, COUT), jnp.float32)] * 2,
    )(acc)


def _norm_body(x_ref, a_ref, b_ref, o_ref):
    x = x_ref[...].astype(jnp.float32)
    o_ref[...] = jnp.maximum(x * a_ref[...] + b_ref[...], 0.0)


def _normalize(acc, a, b):
    nb = N_PTS // BN_EW
    return pl.pallas_call(
        _norm_body,
        grid=(nb,),
        in_specs=[
            pl.BlockSpec((BN_EW, COUT), lambda n: (n, 0)),
            pl.BlockSpec((1, COUT), lambda n: (0, 0)),
            pl.BlockSpec((1, COUT), lambda n: (0, 0)),
        ],
        out_specs=pl.BlockSpec((BN_EW, COUT), lambda n: (n, 0)),
        out_shape=jax.ShapeDtypeStruct((N_PTS, COUT), jnp.float32),
    )(acc, a, b)


def kernel(coords, feats, rules, weight, bias, gamma, beta):
    y = _dense_matmuls(feats, weight)
    y = y.reshape(K_OFF * N_PTS, COUT)

    offs = (jnp.arange(K_OFF, dtype=jnp.int32) * N_PTS)[:, None]
    in_flat = (rules[:, 0, :] + offs).reshape(-1)
    out_flat = rules[:, 1, :].reshape(-1)
    zrs = jnp.zeros((STRIPE, COUT), jnp.bfloat16)

    acc = _sc_scatter(y, in_flat, out_flat, zrs)

    s, q = _channel_stats(acc)
    mean = s / N_PTS
    var = q / N_PTS - mean * mean
    a = (gamma / jnp.sqrt(var[0] + 1e-5))[None]
    b = (beta + (bias - mean[0]) * a[0])[None]
    out = _normalize(acc, a, b)
    return (coords, out)


# speedup vs baseline: 1.3069x; 1.0058x over previous
import functools

import jax
import jax.numpy as jnp
from jax import lax
from jax.experimental import pallas as pl
from jax.experimental.pallas import tpu as pltpu
from jax.experimental.pallas import tpu_sc as plsc

N_PTS = 50000
CIN = 128
COUT = 128
K_OFF = 9

NC = 2
NS = 16
ROWS_PER_SC = 25000
ACC_ROWS = 25088
STRIPE = ACC_ROWS // NS
DUMMY = ROWS_PER_SC
CHUNK = 128
SUP = 8
SUP_PAIRS = SUP * CHUNK
PAIRS_REAL = K_OFF * N_PTS
PAIRS_SPAN = 458752
SUPS_PER_TILE = PAIRS_SPAN // (NS * SUP_PAIRS)

BN_MM = 2000
BN_EW = 2000


def _mm_body(x_ref, w_ref, y_ref):
    y = jnp.dot(x_ref[...], w_ref[0], preferred_element_type=jnp.float32)
    y_ref[0] = y.astype(jnp.bfloat16)


def _dense_matmuls(feats, weight):
    nb = N_PTS // BN_MM
    return pl.pallas_call(
        _mm_body,
        grid=(nb, K_OFF),
        in_specs=[
            pl.BlockSpec((BN_MM, CIN), lambda n, k: (n, 0)),
            pl.BlockSpec((1, CIN, COUT), lambda n, k: (k, 0, 0)),
        ],
        out_specs=pl.BlockSpec((1, BN_MM, COUT), lambda n, k: (k, n, 0)),
        out_shape=jax.ShapeDtypeStruct((K_OFF, N_PTS, COUT), jnp.bfloat16),
    )(feats, weight)


def _prep_body(r_ref, if_ref, of_ref):
    k = pl.program_id(0)
    r = r_ref[0]
    if_ref[0] = r[0:1, :] + k * N_PTS
    of_ref[0] = r[1:2, :]


def _prep_indices(rules):
    return pl.pallas_call(
        _prep_body,
        grid=(K_OFF,),
        in_specs=[
            pl.BlockSpec((1, 2, N_PTS), lambda k: (k, 0, 0)),
        ],
        out_specs=[
            pl.BlockSpec((1, 1, N_PTS), lambda k: (k, 0, 0)),
            pl.BlockSpec((1, 1, N_PTS), lambda k: (k, 0, 0)),
        ],
        out_shape=[jax.ShapeDtypeStruct((K_OFF, 1, N_PTS), jnp.int32)] * 2,
    )(rules)


def _sc_scatter(y, in_flat, out_flat, zrs):
    mesh = plsc.VectorSubcoreMesh(core_axis_name="c", subcore_axis_name="s")

    @functools.partial(
        pl.kernel,
        mesh=mesh,
        compiler_params=pltpu.CompilerParams(use_tc_tiling_on_sc=False),
        out_type=jax.ShapeDtypeStruct((N_PTS, COUT), jnp.bfloat16),
        scratch_types=[
            pltpu.VMEM((SUP_PAIRS,), jnp.int32),
            pltpu.VMEM((SUP_PAIRS,), jnp.int32),
            pltpu.VMEM((SUP, CHUNK), jnp.int32),
            pltpu.VMEM((2 * CHUNK, COUT), jnp.bfloat16),
            pltpu.VMEM_SHARED((ACC_ROWS, COUT), jnp.bfloat16),
            pltpu.SemaphoreType.DMA,
            pltpu.SemaphoreType.DMA,
        ],
    )
    def scatter_kernel(y_h, inf_h, outf_h, zrs_h, o_h,
                       idx_v, oraw_v, loc_v, rows_v, acc_s, sem0, sem1):
        cid = lax.axis_index("c")
        sid = lax.axis_index("s")
        lo = cid * ROWS_PER_SC

        pltpu.sync_copy(zrs_h, acc_s.at[pl.ds(sid * STRIPE, STRIPE)])
        plsc.subcore_barrier()

        lane = lax.iota(jnp.int32, 16)

        def sup_body(ci, _):
            b_n = (sid * SUPS_PER_TILE + ci) * SUP_PAIRS
            b_r = pl.multiple_of(
                jnp.minimum(b_n, PAIRS_REAL - SUP_PAIRS), 8)
            diff = b_n - b_r
            pltpu.sync_copy(inf_h.at[pl.ds(b_r, SUP_PAIRS)], idx_v)
            pltpu.sync_copy(outf_h.at[pl.ds(b_r, SUP_PAIRS)], oraw_v)
            for jr in range(SUP):
                for jc in range(CHUNK // 16):
                    off = jr * CHUNK + jc * 16
                    o = oraw_v[pl.ds(off, 16)]
                    keep = ((o >= lo) & (o < lo + ROWS_PER_SC)
                            & (lane + off >= diff))
                    loc_v[jr, pl.ds(jc * 16, 16)] = (
                        jnp.where(keep, o - lo, DUMMY))

            def gather(j):
                slot = (j % 2) * CHUNK
                return pltpu.async_copy(
                    y_h.at[idx_v.at[pl.ds(j * CHUNK, CHUNK)]],
                    rows_v.at[pl.ds(slot, CHUNK)],
                    sem0 if j % 2 == 0 else sem1)

            cp = gather(0)
            for j in range(SUP):
                nxt = gather(j + 1) if j + 1 < SUP else None
                cp.wait()
                pltpu.sync_copy(
                    rows_v.at[pl.ds((j % 2) * CHUNK, CHUNK)],
                    acc_s.at[loc_v.at[j]], add=True)
                cp = nxt
            return 0

        lax.fori_loop(0, SUPS_PER_TILE, sup_body, 0)
        plsc.subcore_barrier()

        @pl.when(sid < NS - 1)
        def _():
            pltpu.sync_copy(
                acc_s.at[pl.ds(sid * STRIPE, STRIPE)],
                o_h.at[pl.ds(lo + sid * STRIPE, STRIPE)])

        @pl.when(sid == NS - 1)
        def _():
            tail = ROWS_PER_SC - (NS - 1) * STRIPE
            pltpu.sync_copy(
                acc_s.at[pl.ds((NS - 1) * STRIPE, tail)],
                o_h.at[pl.ds(lo + (NS - 1) * STRIPE, tail)])

    return scatter_kernel(y, in_flat, out_flat, zrs)


def _stats_body(x_ref, s_ref, q_ref):
    x = x_ref[...].astype(jnp.float32)
    s = jnp.sum(x, axis=0, keepdims=True)
    q = jnp.sum(x * x, axis=0, keepdims=True)

    @pl.when(pl.program_id(0) == 0)
    def _():
        s_ref[...] = s
        q_ref[...] = q

    @pl.when(pl.program_id(0) != 0)
    def _():
        s_ref[...] += s
        q_ref[...] += q


def _channel_stats(acc):
    nb = N_PTS // BN_EW
    one = pl.BlockSpec((1, COUT), lambda n: (0, 0))
    return pl.pallas_call(
        _stats_body,
        grid=(nb,),
        in_specs=[pl.BlockSpec((BN_EW, COUT), lambda n: (n, 0))],
        out_specs=[one, one],
        out_shape=[jax.ShapeDtypeStruct((1, COUT), jnp.float32)] * 2,
    )(acc)


def _norm_body(x_ref, a_ref, b_ref, o_ref):
    x = x_ref[...].astype(jnp.float32)
    o_ref[...] = jnp.maximum(x * a_ref[...] + b_ref[...], 0.0)


def _normalize(acc, a, b):
    nb = N_PTS // BN_EW
    return pl.pallas_call(
        _norm_body,
        grid=(nb,),
        in_specs=[
            pl.BlockSpec((BN_EW, COUT), lambda n: (n, 0)),
            pl.BlockSpec((1, COUT), lambda n: (0, 0)),
            pl.BlockSpec((1, COUT), lambda n: (0, 0)),
        ],
        out_specs=pl.BlockSpec((BN_EW, COUT), lambda n: (n, 0)),
        out_shape=jax.ShapeDtypeStruct((N_PTS, COUT), jnp.float32),
    )(acc, a, b)


def kernel(coords, feats, rules, weight, bias, gamma, beta):
    y = _dense_matmuls(feats, weight)
    y = y.reshape(K_OFF * N_PTS, COUT)

    in_flat, out_flat = _prep_indices(rules)
    in_flat = in_flat.reshape(-1)
    out_flat = out_flat.reshape(-1)
    zrs = jnp.zeros((STRIPE, COUT), jnp.bfloat16)

    acc = _sc_scatter(y, in_flat, out_flat, zrs)

    s, q = _channel_stats(acc)
    mean = s / N_PTS
    var = q / N_PTS - mean * mean
    a = (gamma / jnp.sqrt(var[0] + 1e-5))[None]
    b = (beta + (bias - mean[0]) * a[0])[None]
    out = _normalize(acc, a, b)
    return (coords, out)


# speedup vs baseline: 1.3965x; 1.0686x over previous
import functools

import jax
import jax.numpy as jnp
from jax import lax
from jax.experimental import pallas as pl
from jax.experimental.pallas import tpu as pltpu
from jax.experimental.pallas import tpu_sc as plsc

N_PTS = 50000
CIN = 128
COUT = 128
K_OFF = 9

NC = 2
NS = 16
ROWS_PER_SC = 25000
ACC_ROWS = 25088
STRIPE = ACC_ROWS // NS
DUMMY = ROWS_PER_SC
CHUNK = 128
SUP = 8
SUP_PAIRS = SUP * CHUNK
PAIRS_REAL = K_OFF * N_PTS
PAIRS_SPAN = 458752
SUPS_PER_TILE = PAIRS_SPAN // (NS * SUP_PAIRS)

BN_MM = 5000
BN_EW = 2000


def _mm_body(x_ref, w_ref, y_ref):
    y = jnp.dot(x_ref[...], w_ref[0], preferred_element_type=jnp.float32)
    y_ref[0] = y.astype(jnp.bfloat16)


def _dense_matmuls(feats, weight):
    nb = N_PTS // BN_MM
    return pl.pallas_call(
        _mm_body,
        grid=(nb, K_OFF),
        in_specs=[
            pl.BlockSpec((BN_MM, CIN), lambda n, k: (n, 0)),
            pl.BlockSpec((1, CIN, COUT), lambda n, k: (k, 0, 0)),
        ],
        out_specs=pl.BlockSpec((1, BN_MM, COUT), lambda n, k: (k, n, 0)),
        out_shape=jax.ShapeDtypeStruct((K_OFF, N_PTS, COUT), jnp.bfloat16),
    )(feats, weight)


def _prep_body(r_ref, if_ref, of_ref):
    k = pl.program_id(0)
    r = r_ref[0]
    if_ref[0] = r[0:1, :] + k * N_PTS
    of_ref[0] = r[1:2, :]


def _prep_indices(rules):
    return pl.pallas_call(
        _prep_body,
        grid=(K_OFF,),
        in_specs=[
            pl.BlockSpec((1, 2, N_PTS), lambda k: (k, 0, 0)),
        ],
        out_specs=[
            pl.BlockSpec((1, 1, N_PTS), lambda k: (k, 0, 0)),
            pl.BlockSpec((1, 1, N_PTS), lambda k: (k, 0, 0)),
        ],
        out_shape=[jax.ShapeDtypeStruct((K_OFF, 1, N_PTS), jnp.int32)] * 2,
    )(rules)


def _sc_scatter(y, in_flat, out_flat, zrs):
    mesh = plsc.VectorSubcoreMesh(core_axis_name="c", subcore_axis_name="s")

    @functools.partial(
        pl.kernel,
        mesh=mesh,
        compiler_params=pltpu.CompilerParams(use_tc_tiling_on_sc=False),
        out_type=jax.ShapeDtypeStruct((N_PTS, COUT), jnp.bfloat16),
        scratch_types=[
            pltpu.VMEM((SUP_PAIRS,), jnp.int32),
            pltpu.VMEM((SUP_PAIRS,), jnp.int32),
            pltpu.VMEM((SUP, CHUNK), jnp.int32),
            pltpu.VMEM((2 * CHUNK, COUT), jnp.bfloat16),
            pltpu.VMEM_SHARED((ACC_ROWS, COUT), jnp.bfloat16),
            pltpu.SemaphoreType.DMA,
            pltpu.SemaphoreType.DMA,
        ],
    )
    def scatter_kernel(y_h, inf_h, outf_h, zrs_h, o_h,
                       idx_v, oraw_v, loc_v, rows_v, acc_s, sem0, sem1):
        cid = lax.axis_index("c")
        sid = lax.axis_index("s")
        lo = cid * ROWS_PER_SC

        pltpu.sync_copy(zrs_h, acc_s.at[pl.ds(sid * STRIPE, STRIPE)])
        plsc.subcore_barrier()

        lane = lax.iota(jnp.int32, 16)

        def sup_body(ci, _):
            b_n = (sid * SUPS_PER_TILE + ci) * SUP_PAIRS
            b_r = pl.multiple_of(
                jnp.minimum(b_n, PAIRS_REAL - SUP_PAIRS), 8)
            diff = b_n - b_r
            pltpu.sync_copy(inf_h.at[pl.ds(b_r, SUP_PAIRS)], idx_v)
            pltpu.sync_copy(outf_h.at[pl.ds(b_r, SUP_PAIRS)], oraw_v)
            for jr in range(SUP):
                for jc in range(CHUNK // 16):
                    off = jr * CHUNK + jc * 16
                    o = oraw_v[pl.ds(off, 16)]
                    keep = ((o >= lo) & (o < lo + ROWS_PER_SC)
                            & (lane + off >= diff))
                    loc_v[jr, pl.ds(jc * 16, 16)] = (
                        jnp.where(keep, o - lo, DUMMY))

            def gather(j):
                slot = (j % 2) * CHUNK
                return pltpu.async_copy(
                    y_h.at[idx_v.at[pl.ds(j * CHUNK, CHUNK)]],
                    rows_v.at[pl.ds(slot, CHUNK)],
                    sem0 if j % 2 == 0 else sem1)

            cp = gather(0)
            for j in range(SUP):
                nxt = gather(j + 1) if j + 1 < SUP else None
                cp.wait()
                pltpu.sync_copy(
                    rows_v.at[pl.ds((j % 2) * CHUNK, CHUNK)],
                    acc_s.at[loc_v.at[j]], add=True)
                cp = nxt
            return 0

        lax.fori_loop(0, SUPS_PER_TILE, sup_body, 0)
        plsc.subcore_barrier()

        @pl.when(sid < NS - 1)
        def _():
            pltpu.sync_copy(
                acc_s.at[pl.ds(sid * STRIPE, STRIPE)],
                o_h.at[pl.ds(lo + sid * STRIPE, STRIPE)])

        @pl.when(sid == NS - 1)
        def _():
            tail = ROWS_PER_SC - (NS - 1) * STRIPE
            pltpu.sync_copy(
                acc_s.at[pl.ds((NS - 1) * STRIPE, tail)],
                o_h.at[pl.ds(lo + (NS - 1) * STRIPE, tail)])

    return scatter_kernel(y, in_flat, out_flat, zrs)


def _stats_body(x_ref, s_ref, q_ref):
    x = x_ref[...].astype(jnp.float32)
    s = jnp.sum(x, axis=0, keepdims=True)
    q = jnp.sum(x * x, axis=0, keepdims=True)

    @pl.when(pl.program_id(0) == 0)
    def _():
        s_ref[...] = s
        q_ref[...] = q

    @pl.when(pl.program_id(0) != 0)
    def _():
        s_ref[...] += s
        q_ref[...] += q


def _channel_stats(acc):
    nb = N_PTS // BN_EW
    one = pl.BlockSpec((1, COUT), lambda n: (0, 0))
    return pl.pallas_call(
        _stats_body,
        grid=(nb,),
        in_specs=[pl.BlockSpec((BN_EW, COUT), lambda n: (n, 0))],
        out_specs=[one, one],
        out_shape=[jax.ShapeDtypeStruct((1, COUT), jnp.float32)] * 2,
    )(acc)


def _norm_body(x_ref, a_ref, b_ref, o_ref):
    x = x_ref[...].astype(jnp.float32)
    o_ref[...] = jnp.maximum(x * a_ref[...] + b_ref[...], 0.0)


def _normalize(acc, a, b):
    nb = N_PTS // BN_EW
    return pl.pallas_call(
        _norm_body,
        grid=(nb,),
        in_specs=[
            pl.BlockSpec((BN_EW, COUT), lambda n: (n, 0)),
            pl.BlockSpec((1, COUT), lambda n: (0, 0)),
            pl.BlockSpec((1, COUT), lambda n: (0, 0)),
        ],
        out_specs=pl.BlockSpec((BN_EW, COUT), lambda n: (n, 0)),
        out_shape=jax.ShapeDtypeStruct((N_PTS, COUT), jnp.float32),
    )(acc, a, b)


def kernel(coords, feats, rules, weight, bias, gamma, beta):
    y = _dense_matmuls(feats, weight)
    y = y.reshape(K_OFF * N_PTS, COUT)

    in_flat, out_flat = _prep_indices(rules)
    in_flat = in_flat.reshape(-1)
    out_flat = out_flat.reshape(-1)
    zrs = jnp.zeros((STRIPE, COUT), jnp.bfloat16)

    acc = _sc_scatter(y, in_flat, out_flat, zrs)

    s, q = _channel_stats(acc)
    mean = s / N_PTS
    var = q / N_PTS - mean * mean
    a = (gamma / jnp.sqrt(var[0] + 1e-5))[None]
    b = (beta + (bias - mean[0]) * a[0])[None]
    out = _normalize(acc, a, b)
    return (coords, out)


# speedup vs baseline: 1.4194x; 1.0164x over previous
import functools

import jax
import jax.numpy as jnp
from jax import lax
from jax.experimental import pallas as pl
from jax.experimental.pallas import tpu as pltpu
from jax.experimental.pallas import tpu_sc as plsc

N_PTS = 50000
CIN = 128
COUT = 128
K_OFF = 9

NC = 2
NS = 16
ROWS_PER_SC = 25000
ACC_ROWS = 25088
STRIPE = ACC_ROWS // NS
DUMMY = ROWS_PER_SC
CHUNK = 128
SUP = 8
SUP_PAIRS = SUP * CHUNK
PAIRS_REAL = K_OFF * N_PTS
PAIRS_SPAN = 458752
SUPS_PER_TILE = PAIRS_SPAN // (NS * SUP_PAIRS)

BN_MM = 5000
BN_EW = 5000


def _mm_body(x_ref, w_ref, y_ref):
    y = jnp.dot(x_ref[...], w_ref[0], preferred_element_type=jnp.float32)
    y_ref[0] = y.astype(jnp.bfloat16)


def _dense_matmuls(feats, weight):
    nb = N_PTS // BN_MM
    return pl.pallas_call(
        _mm_body,
        grid=(nb, K_OFF),
        in_specs=[
            pl.BlockSpec((BN_MM, CIN), lambda n, k: (n, 0)),
            pl.BlockSpec((1, CIN, COUT), lambda n, k: (k, 0, 0)),
        ],
        out_specs=pl.BlockSpec((1, BN_MM, COUT), lambda n, k: (k, n, 0)),
        out_shape=jax.ShapeDtypeStruct((K_OFF, N_PTS, COUT), jnp.bfloat16),
    )(feats, weight)


def _prep_body(r_ref, if_ref, of_ref):
    k = pl.program_id(0)
    r = r_ref[0]
    if_ref[0] = r[0:1, :] + k * N_PTS
    of_ref[0] = r[1:2, :]


def _prep_indices(rules):
    return pl.pallas_call(
        _prep_body,
        grid=(K_OFF,),
        in_specs=[
            pl.BlockSpec((1, 2, N_PTS), lambda k: (k, 0, 0)),
        ],
        out_specs=[
            pl.BlockSpec((1, 1, N_PTS), lambda k: (k, 0, 0)),
            pl.BlockSpec((1, 1, N_PTS), lambda k: (k, 0, 0)),
        ],
        out_shape=[jax.ShapeDtypeStruct((K_OFF, 1, N_PTS), jnp.int32)] * 2,
    )(rules)


def _sc_scatter(y, in_flat, out_flat, zrs):
    mesh = plsc.VectorSubcoreMesh(core_axis_name="c", subcore_axis_name="s")

    @functools.partial(
        pl.kernel,
        mesh=mesh,
        compiler_params=pltpu.CompilerParams(use_tc_tiling_on_sc=False),
        out_type=jax.ShapeDtypeStruct((N_PTS, COUT), jnp.bfloat16),
        scratch_types=[
            pltpu.VMEM((SUP_PAIRS,), jnp.int32),
            pltpu.VMEM((SUP_PAIRS,), jnp.int32),
            pltpu.VMEM((SUP, CHUNK), jnp.int32),
            pltpu.VMEM((2 * CHUNK, COUT), jnp.bfloat16),
            pltpu.VMEM_SHARED((ACC_ROWS, COUT), jnp.bfloat16),
            pltpu.SemaphoreType.DMA,
            pltpu.SemaphoreType.DMA,
        ],
    )
    def scatter_kernel(y_h, inf_h, outf_h, zrs_h, o_h,
                       idx_v, oraw_v, loc_v, rows_v, acc_s, sem0, sem1):
        cid = lax.axis_index("c")
        sid = lax.axis_index("s")
        lo = cid * ROWS_PER_SC

        pltpu.sync_copy(zrs_h, acc_s.at[pl.ds(sid * STRIPE, STRIPE)])
        plsc.subcore_barrier()

        lane = lax.iota(jnp.int32, 16)

        def sup_body(ci, _):
            b_n = (sid * SUPS_PER_TILE + ci) * SUP_PAIRS
            b_r = pl.multiple_of(
                jnp.minimum(b_n, PAIRS_REAL - SUP_PAIRS), 8)
            diff = b_n - b_r
            pltpu.sync_copy(inf_h.at[pl.ds(b_r, SUP_PAIRS)], idx_v)
            pltpu.sync_copy(outf_h.at[pl.ds(b_r, SUP_PAIRS)], oraw_v)
            for jr in range(SUP):
                for jc in range(CHUNK // 16):
                    off = jr * CHUNK + jc * 16
                    o = oraw_v[pl.ds(off, 16)]
                    keep = ((o >= lo) & (o < lo + ROWS_PER_SC)
                            & (lane + off >= diff))
                    loc_v[jr, pl.ds(jc * 16, 16)] = (
                        jnp.where(keep, o - lo, DUMMY))

            def gather(j):
                slot = (j % 2) * CHUNK
                return pltpu.async_copy(
                    y_h.at[idx_v.at[pl.ds(j * CHUNK, CHUNK)]],
                    rows_v.at[pl.ds(slot, CHUNK)],
                    sem0 if j % 2 == 0 else sem1)

            cp = gather(0)
            for j in range(SUP):
                nxt = gather(j + 1) if j + 1 < SUP else None
                cp.wait()
                pltpu.sync_copy(
                    rows_v.at[pl.ds((j % 2) * CHUNK, CHUNK)],
                    acc_s.at[loc_v.at[j]], add=True)
                cp = nxt
            return 0

        lax.fori_loop(0, SUPS_PER_TILE, sup_body, 0)
        plsc.subcore_barrier()

        @pl.when(sid < NS - 1)
        def _():
            pltpu.sync_copy(
                acc_s.at[pl.ds(sid * STRIPE, STRIPE)],
                o_h.at[pl.ds(lo + sid * STRIPE, STRIPE)])

        @pl.when(sid == NS - 1)
        def _():
            tail = ROWS_PER_SC - (NS - 1) * STRIPE
            pltpu.sync_copy(
                acc_s.at[pl.ds((NS - 1) * STRIPE, tail)],
                o_h.at[pl.ds(lo + (NS - 1) * STRIPE, tail)])

    return scatter_kernel(y, in_flat, out_flat, zrs)


def _stats_body(x_ref, s_ref, q_ref):
    x = x_ref[...].astype(jnp.float32)
    s = jnp.sum(x, axis=0, keepdims=True)
    q = jnp.sum(x * x, axis=0, keepdims=True)

    @pl.when(pl.program_id(0) == 0)
    def _():
        s_ref[...] = s
        q_ref[...] = q

    @pl.when(pl.program_id(0) != 0)
    def _():
        s_ref[...] += s
        q_ref[...] += q


def _channel_stats(acc):
    nb = N_PTS // BN_EW
    one = pl.BlockSpec((1, COUT), lambda n: (0, 0))
    return pl.pallas_call(
        _stats_body,
        grid=(nb,),
        in_specs=[pl.BlockSpec((BN_EW, COUT), lambda n: (n, 0))],
        out_specs=[one, one],
        out_shape=[jax.ShapeDtypeStruct((1, COUT), jnp.float32)] * 2,
    )(acc)


def _norm_body(x_ref, a_ref, b_ref, o_ref):
    x = x_ref[...].astype(jnp.float32)
    o_ref[...] = jnp.maximum(x * a_ref[...] + b_ref[...], 0.0)


def _normalize(acc, a, b):
    nb = N_PTS // BN_EW
    return pl.pallas_call(
        _norm_body,
        grid=(nb,),
        in_specs=[
            pl.BlockSpec((BN_EW, COUT), lambda n: (n, 0)),
            pl.BlockSpec((1, COUT), lambda n: (0, 0)),
            pl.BlockSpec((1, COUT), lambda n: (0, 0)),
        ],
        out_specs=pl.BlockSpec((BN_EW, COUT), lambda n: (n, 0)),
        out_shape=jax.ShapeDtypeStruct((N_PTS, COUT), jnp.float32),
    )(acc, a, b)


def kernel(coords, feats, rules, weight, bias, gamma, beta):
    y = _dense_matmuls(feats, weight)
    y = y.reshape(K_OFF * N_PTS, COUT)

    in_flat, out_flat = _prep_indices(rules)
    in_flat = in_flat.reshape(-1)
    out_flat = out_flat.reshape(-1)
    zrs = jnp.zeros((STRIPE, COUT), jnp.bfloat16)

    acc = _sc_scatter(y, in_flat, out_flat, zrs)

    s, q = _channel_stats(acc)
    mean = s / N_PTS
    var = q / N_PTS - mean * mean
    a = (gamma / jnp.sqrt(var[0] + 1e-5))[None]
    b = (beta + (bias - mean[0]) * a[0])[None]
    out = _normalize(acc, a, b)
    return (coords, out)


# speedup vs baseline: 1.4880x; 1.0483x over previous
import functools

import jax
import jax.numpy as jnp
from jax import lax
from jax.experimental import pallas as pl
from jax.experimental.pallas import tpu as pltpu
from jax.experimental.pallas import tpu_sc as plsc

N_PTS = 50000
CIN = 128
COUT = 128
K_OFF = 9

NC = 2
NS = 16
ROWS_PER_SC = 25000
ACC_ROWS = 25088
STRIPE = ACC_ROWS // NS
DUMMY = ROWS_PER_SC
CHUNK = 128
SUP = 8
SUP_PAIRS = SUP * CHUNK
PAIRS_REAL = K_OFF * N_PTS
PAIRS_SPAN = 458752
SUPS_PER_TILE = PAIRS_SPAN // (NS * SUP_PAIRS)

BN_MM = 10000
BN_EW = 10000


def _mm_body(x_ref, w_ref, y_ref):
    y = jnp.dot(x_ref[...], w_ref[0], preferred_element_type=jnp.float32)
    y_ref[0] = y.astype(jnp.bfloat16)


def _dense_matmuls(feats, weight):
    nb = N_PTS // BN_MM
    return pl.pallas_call(
        _mm_body,
        grid=(nb, K_OFF),
        in_specs=[
            pl.BlockSpec((BN_MM, CIN), lambda n, k: (n, 0)),
            pl.BlockSpec((1, CIN, COUT), lambda n, k: (k, 0, 0)),
        ],
        out_specs=pl.BlockSpec((1, BN_MM, COUT), lambda n, k: (k, n, 0)),
        out_shape=jax.ShapeDtypeStruct((K_OFF, N_PTS, COUT), jnp.bfloat16),
    )(feats, weight)


def _prep_body(r_ref, if_ref, of_ref):
    k = pl.program_id(0)
    r = r_ref[0]
    if_ref[0] = r[0:1, :] + k * N_PTS
    of_ref[0] = r[1:2, :]


def _prep_indices(rules):
    return pl.pallas_call(
        _prep_body,
        grid=(K_OFF,),
        in_specs=[
            pl.BlockSpec((1, 2, N_PTS), lambda k: (k, 0, 0)),
        ],
        out_specs=[
            pl.BlockSpec((1, 1, N_PTS), lambda k: (k, 0, 0)),
            pl.BlockSpec((1, 1, N_PTS), lambda k: (k, 0, 0)),
        ],
        out_shape=[jax.ShapeDtypeStruct((K_OFF, 1, N_PTS), jnp.int32)] * 2,
    )(rules)


def _sc_scatter(y, in_flat, out_flat, zrs):
    mesh = plsc.VectorSubcoreMesh(core_axis_name="c", subcore_axis_name="s")

    @functools.partial(
        pl.kernel,
        mesh=mesh,
        compiler_params=pltpu.CompilerParams(use_tc_tiling_on_sc=False),
        out_type=jax.ShapeDtypeStruct((N_PTS, COUT), jnp.bfloat16),
        scratch_types=[
            pltpu.VMEM((SUP_PAIRS,), jnp.int32),
            pltpu.VMEM((SUP_PAIRS,), jnp.int32),
            pltpu.VMEM((SUP, CHUNK), jnp.int32),
            pltpu.VMEM((2 * CHUNK, COUT), jnp.bfloat16),
            pltpu.VMEM_SHARED((ACC_ROWS, COUT), jnp.bfloat16),
            pltpu.SemaphoreType.DMA,
            pltpu.SemaphoreType.DMA,
        ],
    )
    def scatter_kernel(y_h, inf_h, outf_h, zrs_h, o_h,
                       idx_v, oraw_v, loc_v, rows_v, acc_s, sem0, sem1):
        cid = lax.axis_index("c")
        sid = lax.axis_index("s")
        lo = cid * ROWS_PER_SC

        pltpu.sync_copy(zrs_h, acc_s.at[pl.ds(sid * STRIPE, STRIPE)])
        plsc.subcore_barrier()

        lane = lax.iota(jnp.int32, 16)

        def sup_body(ci, _):
            b_n = (sid * SUPS_PER_TILE + ci) * SUP_PAIRS
            b_r = pl.multiple_of(
                jnp.minimum(b_n, PAIRS_REAL - SUP_PAIRS), 8)
            diff = b_n - b_r
            pltpu.sync_copy(inf_h.at[pl.ds(b_r, SUP_PAIRS)], idx_v)
            pltpu.sync_copy(outf_h.at[pl.ds(b_r, SUP_PAIRS)], oraw_v)
            for jr in range(SUP):
                for jc in range(CHUNK // 16):
                    off = jr * CHUNK + jc * 16
                    o = oraw_v[pl.ds(off, 16)]
                    keep = ((o >= lo) & (o < lo + ROWS_PER_SC)
                            & (lane + off >= diff))
                    loc_v[jr, pl.ds(jc * 16, 16)] = (
                        jnp.where(keep, o - lo, DUMMY))

            def gather(j):
                slot = (j % 2) * CHUNK
                return pltpu.async_copy(
                    y_h.at[idx_v.at[pl.ds(j * CHUNK, CHUNK)]],
                    rows_v.at[pl.ds(slot, CHUNK)],
                    sem0 if j % 2 == 0 else sem1)

            cp = gather(0)
            for j in range(SUP):
                nxt = gather(j + 1) if j + 1 < SUP else None
                cp.wait()
                pltpu.sync_copy(
                    rows_v.at[pl.ds((j % 2) * CHUNK, CHUNK)],
                    acc_s.at[loc_v.at[j]], add=True)
                cp = nxt
            return 0

        lax.fori_loop(0, SUPS_PER_TILE, sup_body, 0)
        plsc.subcore_barrier()

        @pl.when(sid < NS - 1)
        def _():
            pltpu.sync_copy(
                acc_s.at[pl.ds(sid * STRIPE, STRIPE)],
                o_h.at[pl.ds(lo + sid * STRIPE, STRIPE)])

        @pl.when(sid == NS - 1)
        def _():
            tail = ROWS_PER_SC - (NS - 1) * STRIPE
            pltpu.sync_copy(
                acc_s.at[pl.ds((NS - 1) * STRIPE, tail)],
                o_h.at[pl.ds(lo + (NS - 1) * STRIPE, tail)])

    return scatter_kernel(y, in_flat, out_flat, zrs)


def _stats_body(x_ref, s_ref, q_ref):
    x = x_ref[...].astype(jnp.float32)
    s = jnp.sum(x, axis=0, keepdims=True)
    q = jnp.sum(x * x, axis=0, keepdims=True)

    @pl.when(pl.program_id(0) == 0)
    def _():
        s_ref[...] = s
        q_ref[...] = q

    @pl.when(pl.program_id(0) != 0)
    def _():
        s_ref[...] += s
        q_ref[...] += q


def _channel_stats(acc):
    nb = N_PTS // BN_EW
    one = pl.BlockSpec((1, COUT), lambda n: (0, 0))
    return pl.pallas_call(
        _stats_body,
        grid=(nb,),
        in_specs=[pl.BlockSpec((BN_EW, COUT), lambda n: (n, 0))],
        out_specs=[one, one],
        out_shape=[jax.ShapeDtypeStruct((1, COUT), jnp.float32)] * 2,
    )(acc)


def _norm_body(x_ref, a_ref, b_ref, o_ref):
    x = x_ref[...].astype(jnp.float32)
    o_ref[...] = jnp.maximum(x * a_ref[...] + b_ref[...], 0.0)


def _normalize(acc, a, b):
    nb = N_PTS // BN_EW
    return pl.pallas_call(
        _norm_body,
        grid=(nb,),
        in_specs=[
            pl.BlockSpec((BN_EW, COUT), lambda n: (n, 0)),
            pl.BlockSpec((1, COUT), lambda n: (0, 0)),
            pl.BlockSpec((1, COUT), lambda n: (0, 0)),
        ],
        out_specs=pl.BlockSpec((BN_EW, COUT), lambda n: (n, 0)),
        out_shape=jax.ShapeDtypeStruct((N_PTS, COUT), jnp.float32),
    )(acc, a, b)


def kernel(coords, feats, rules, weight, bias, gamma, beta):
    y = _dense_matmuls(feats, weight)
    y = y.reshape(K_OFF * N_PTS, COUT)

    in_flat, out_flat = _prep_indices(rules)
    in_flat = in_flat.reshape(-1)
    out_flat = out_flat.reshape(-1)
    zrs = jnp.zeros((STRIPE, COUT), jnp.bfloat16)

    acc = _sc_scatter(y, in_flat, out_flat, zrs)

    s, q = _channel_stats(acc)
    mean = s / N_PTS
    var = q / N_PTS - mean * mean
    a = (gamma / jnp.sqrt(var[0] + 1e-5))[None]
    b = (beta + (bias - mean[0]) * a[0])[None]
    out = _normalize(acc, a, b)
    return (coords, out)
